# Initial kernel scaffold; baseline (speedup 1.0000x reference)
import jax
import jax.numpy as jnp
from jax.experimental import pallas as pl


def kernel(x, edge_index, W, b):
    raise NotImplementedError("write your pallas kernel here")


# speedup vs baseline: 24.7056x; 24.7056x over previous
import dataclasses
import functools

import jax
import jax.numpy as jnp
from jax import lax
from jax.experimental import pallas as pl
from jax.experimental.pallas import tpu as pltpu
from jax.experimental.pallas import tpu_sc as plsc

NC = 2
NS = 16
NW = NC * NS
LANES = 16


def _sc_mesh():
    return plsc.VectorSubcoreMesh(core_axis_name="c", subcore_axis_name="s")


def _sc_compiler_params():
    cp = pltpu.CompilerParams()
    if "needs_layout_passes" in pltpu.CompilerParams.__dataclass_fields__:
        cp = dataclasses.replace(cp, needs_layout_passes=False)
    return cp


def _sc_hist(dst, n_nodes):
    (e,) = dst.shape
    assert e % (NW * LANES) == 0
    epw = e // NW

    @functools.partial(
        pl.kernel,
        out_type=jax.ShapeDtypeStruct((NW, 1, n_nodes), jnp.float32),
        mesh=_sc_mesh(),
        compiler_params=_sc_compiler_params(),
        scratch_types=[
            pltpu.VMEM((epw,), jnp.int32),
            pltpu.VMEM((1, n_nodes), jnp.float32),
        ],
    )
    def k(dst_hbm, out_hbm, idx_v, hist_v):
        cid = lax.axis_index("c")
        sid = lax.axis_index("s")
        wid = cid * NS + sid
        pltpu.sync_copy(dst_hbm.at[pl.ds(wid * epw, epw)], idx_v)

        @pl.loop(0, n_nodes // LANES)
        def _(i):
            hist_v[0, pl.ds(i * LANES, LANES)] = jnp.zeros((LANES,), jnp.float32)

        ones = jnp.ones((LANES,), jnp.float32)
        iz = jnp.zeros((LANES,), jnp.int32)

        @pl.loop(0, epw // LANES)
        def _(i):
            idx = idx_v[pl.ds(i * LANES, LANES)]
            plsc.addupdate_scatter(hist_v, [iz, idx], ones)

        pltpu.sync_copy(hist_v, out_hbm.at[wid])

    return k(dst)


def _tc_dis(hist):
    nw, n = hist.shape

    def body(h_ref, o_ref):
        deg = jnp.sum(h_ref[...], axis=0) + 1.0
        o_ref[...] = lax.rsqrt(deg)[:, None]

    return pl.pallas_call(
        body,
        out_shape=jax.ShapeDtypeStruct((n, 1), jnp.float32),
    )(hist)


def _tc_prescale(x, w, dis, block_rows):
    n, d_in = x.shape
    d_out = w.shape[1]

    def body(x_ref, w_ref, dis_ref, y_ref):
        xw = jnp.dot(x_ref[...], w_ref[...], preferred_element_type=jnp.float32)
        y_ref[...] = xw * dis_ref[...]

    return pl.pallas_call(
        body,
        grid=(n // block_rows,),
        in_specs=[
            pl.BlockSpec((block_rows, d_in), lambda i: (i, 0)),
            pl.BlockSpec((d_in, d_out), lambda i: (0, 0)),
            pl.BlockSpec((block_rows, 1), lambda i: (i, 0)),
        ],
        out_specs=pl.BlockSpec((block_rows, d_out), lambda i: (i, 0)),
        out_shape=jax.ShapeDtypeStruct((n, d_out), jnp.float32),
    )(x, w, dis)


def _sc_edge(src, dst, y):
    (e,) = src.shape
    n, d = y.shape
    ch = 128
    assert e % ch == 0
    n_chunks = e // ch
    n_loops = (n_chunks + NW - 1) // NW
    rpt = (n // NS) // 8 * 8
    tail = n - rpt * NS
    zr = rpt // 3
    assert zr % 8 == 0 and zr * 3 == rpt and tail % 8 == 0 and tail <= zr

    @functools.partial(
        pl.kernel,
        out_type=jax.ShapeDtypeStruct((NC, n, d), jnp.float32),
        mesh=_sc_mesh(),
        compiler_params=_sc_compiler_params(),
        scratch_types=[
            pltpu.VMEM((ch,), jnp.int32),
            pltpu.VMEM((ch,), jnp.int32),
            pltpu.VMEM((ch, d), jnp.float32),
            pltpu.VMEM((zr, d), jnp.float32),
            pltpu.VMEM_SHARED((n, d), jnp.float32),
            pltpu.SemaphoreType.DMA,
        ],
    )
    def k(src_hbm, dst_hbm, y_hbm, out_hbm, si_v, di_v, rows_v, zero_v, acc_sh, sem):
        cid = lax.axis_index("c")
        sid = lax.axis_index("s")
        wid = cid * NS + sid

        @pl.loop(0, zr)
        def _(r):
            @pl.loop(0, d // LANES)
            def _(j):
                zero_v[r, pl.ds(j * LANES, LANES)] = jnp.zeros((LANES,), jnp.float32)

        @pl.loop(0, 3)
        def _(t):
            r0 = sid * rpt + t * zr
            pltpu.sync_copy(zero_v, acc_sh.at[pl.ds(r0, zr)])

        if tail:
            @pl.when(sid == NS - 1)
            def _():
                pltpu.sync_copy(
                    zero_v.at[pl.ds(0, tail)], acc_sh.at[pl.ds(NS * rpt, tail)]
                )

        plsc.subcore_barrier()

        @pl.loop(0, n_loops)
        def _(c):
            chunk = c * NW + wid

            @pl.when(chunk < n_chunks)
            def _():
                base = chunk * ch
                pltpu.sync_copy(src_hbm.at[pl.ds(base, ch)], si_v)
                pltpu.sync_copy(dst_hbm.at[pl.ds(base, ch)], di_v)
                pltpu.async_copy(y_hbm.at[si_v], rows_v, sem).wait()
                pltpu.sync_copy(rows_v, acc_sh.at[di_v], add=True)

        plsc.subcore_barrier()

        r0 = sid * rpt
        pltpu.sync_copy(
            acc_sh.at[pl.ds(r0, rpt)],
            out_hbm.at[cid].at[pl.ds(r0, rpt)],
        )
        if tail:
            @pl.when(sid == NS - 1)
            def _():
                pltpu.sync_copy(
                    acc_sh.at[pl.ds(NS * rpt, tail)],
                    out_hbm.at[cid].at[pl.ds(NS * rpt, tail)],
                )

    return k(src, dst, y)


def _tc_epilogue(partials, y, dis, b, block_rows):
    n, d = y.shape

    def body(p_ref, y_ref, dis_ref, b_ref, o_ref):
        acc = p_ref[0] + p_ref[1] + y_ref[...]
        o_ref[...] = jnp.maximum(acc * dis_ref[...] + b_ref[...], 0.0)

    return pl.pallas_call(
        body,
        grid=(n // block_rows,),
        in_specs=[
            pl.BlockSpec((NC, block_rows, d), lambda i: (0, i, 0)),
            pl.BlockSpec((block_rows, d), lambda i: (i, 0)),
            pl.BlockSpec((block_rows, 1), lambda i: (i, 0)),
            pl.BlockSpec((1, d), lambda i: (0, 0)),
        ],
        out_specs=pl.BlockSpec((block_rows, d), lambda i: (i, 0)),
        out_shape=jax.ShapeDtypeStruct((n, d), jnp.float32),
    )(partials, y, dis, b.reshape(1, d))


@jax.jit
def kernel(x, edge_index, W, b):
    n = x.shape[0]
    src = edge_index[0]
    dst = edge_index[1]
    hist = _sc_hist(dst, n).reshape(NW, n)
    dis = _tc_dis(hist)
    y = _tc_prescale(x, W, dis, block_rows=1000)
    partials = _sc_edge(src, dst, y)
    return _tc_epilogue(partials, y, dis, b, block_rows=1000)
